# baseline (speedup 1.0000x reference)
import functools

import jax
import jax.numpy as jnp
from jax import lax
from jax.experimental import pallas as pl
from jax.experimental.pallas import tpu as pltpu
from jax.experimental.pallas import tpu_sc as plsc

N = 10000
E = 320000
C_IN = 128
C_HID = 256
C_OUT = 128

NC = 2
NS = 16
NW = NC * NS
CHUNK = 128
W128 = 128

N_PAD = 10240
ROWS_PER_TILE = N_PAD // NS
TRASH_ROW = N

_EQ = NW * CHUNK * 8
E_PAD = ((E + _EQ - 1) // _EQ) * _EQ
WTILE_E = E_PAD // NW
WCHUNKS = WTILE_E // CHUNK
TOT_CHUNKS = E_PAD // CHUNK
A_CHUNKS = 32
B_CHUNKS = TOT_CHUNKS // NS - A_CHUNKS
SEG_CHUNKS = 32

_F32 = jnp.float32


def _mesh():
    return plsc.VectorSubcoreMesh(core_axis_name="c", subcore_axis_name="s")


@functools.partial(
    pl.kernel,
    out_type=(
        jax.ShapeDtypeStruct((N_PAD, 16), _F32),
        jax.ShapeDtypeStruct((N_PAD, 16), _F32),
    ),
    mesh=_mesh(),
    scratch_types=[
        pltpu.VMEM((CHUNK, 16), _F32),
        pltpu.VMEM((WCHUNKS, 128), jnp.int32),
        pltpu.VMEM_SHARED((N_PAD, 16), _F32),
    ],
)
def _deg_kernel(dst2d, z16, deg_a, deg_b, ones_v, didx, dacc):
    c = lax.axis_index("c")
    s = lax.axis_index("s")
    wid = s * NC + c
    rows = pl.ds(s * ROWS_PER_TILE, ROWS_PER_TILE)

    pltpu.sync_copy(z16.at[rows], dacc.at[rows])

    def _fill(i, _):
        ones_v[i, :] = jnp.full((16,), 1.0, _F32)
        return 0

    lax.fori_loop(0, CHUNK, _fill, 0)
    pltpu.sync_copy(dst2d.at[pl.ds(wid * WCHUNKS, WCHUNKS)], didx)
    plsc.subcore_barrier()

    def _body(j, _):
        pltpu.sync_copy(ones_v, dacc.at[didx.at[j]], add=True)
        return 0

    lax.fori_loop(0, WCHUNKS, _body, 0)
    plsc.subcore_barrier()

    @pl.when(c == 0)
    def _():
        pltpu.sync_copy(dacc.at[rows], deg_a.at[rows])

    @pl.when(c == 1)
    def _():
        pltpu.sync_copy(dacc.at[rows], deg_b.at[rows])


@functools.partial(
    pl.kernel,
    out_type=(
        jax.ShapeDtypeStruct((N_PAD, W128), _F32),
        jax.ShapeDtypeStruct((N_PAD, W128), _F32),
    ),
    mesh=_mesh(),
    scratch_types=[
        pltpu.VMEM((SEG_CHUNKS * CHUNK,), jnp.int32),
        pltpu.VMEM((SEG_CHUNKS, 128), jnp.int32),
        pltpu.VMEM((CHUNK, W128), _F32),
        pltpu.VMEM((CHUNK, W128), _F32),
        pltpu.VMEM_SHARED((N_PAD, W128), _F32),
        pltpu.SemaphoreType.DMA,
        pltpu.SemaphoreType.DMA,
    ],
)
def _agg_kernel(table, src1d, dst2d, part_a, part_b, sidx, didx, gbuf0, gbuf1,
                acc, semg, sems):
    gbuf = (gbuf0, gbuf1)
    c = lax.axis_index("c")
    s = lax.axis_index("s")
    rows = pl.ds(s * ROWS_PER_TILE, ROWS_PER_TILE)

    pltpu.sync_copy(table.at[rows], acc.at[rows])
    plsc.subcore_barrier()

    def _g_start(j, b):
        pltpu.async_copy(table.at[sidx.at[pl.ds(j * CHUNK, CHUNK)]],
                         gbuf[b], semg)

    def _g_wait(j, b):
        pltpu.make_async_copy(table.at[sidx.at[pl.ds(j * CHUNK, CHUNK)]],
                              gbuf[b], semg).wait()

    def _s_sync(j, b):
        pltpu.sync_copy(gbuf[b], acc.at[didx.at[j]], add=True)

    def _edge_loop(base_chunk, nchunks):
        for seg in range(nchunks // SEG_CHUNKS):
            cb = base_chunk + seg * SEG_CHUNKS
            pltpu.sync_copy(
                src1d.at[pl.ds(cb * CHUNK, SEG_CHUNKS * CHUNK)], sidx)
            pltpu.sync_copy(dst2d.at[pl.ds(cb, SEG_CHUNKS)], didx)

            def _body(p, _):
                j0 = 2 * p
                j1 = j0 + 1
                _g_start(j0, 0)
                _g_start(j1, 1)
                _g_wait(j0, 0)
                _s_sync(j0, 0)
                _g_wait(j1, 1)
                _s_sync(j1, 1)
                return 0

            lax.fori_loop(0, SEG_CHUNKS // 2, _body, 0)

    @pl.when(c == 0)
    def _():
        _edge_loop(s * A_CHUNKS, A_CHUNKS)

    @pl.when(c == 1)
    def _():
        _edge_loop(NS * A_CHUNKS + s * B_CHUNKS, B_CHUNKS)

    plsc.subcore_barrier()

    @pl.when(c == 0)
    def _():
        pltpu.sync_copy(acc.at[rows], part_a.at[rows])

    @pl.when(c == 1)
    def _():
        pltpu.sync_copy(acc.at[rows], part_b.at[rows])


_BLK = 256
_GRID = (N_PAD // _BLK,)


def _dinv_block(dega, degb):
    deg = dega[...] + degb[...] + 1.0
    return lax.rsqrt(deg)[:, 0:1]


def _m1_body(x_ref, w_ref, dega, degb, hlo_ref, hhi_ref):
    dinv = _dinv_block(dega, degb)
    h = jnp.dot(x_ref[...], w_ref[...], preferred_element_type=_F32) * dinv
    hlo_ref[...] = h[:, :W128]
    hhi_ref[...] = h[:, W128:]


def _m2_body(alo0, alo1, ahi0, ahi1, hlo, hhi, dega, degb, b1_ref, w2_ref,
             h2_ref):
    dinv = _dinv_block(dega, degb)
    agg_lo = alo0[...] + alo1[...] - hlo[...]
    agg_hi = ahi0[...] + ahi1[...] - hhi[...]
    agg = jnp.concatenate([agg_lo, agg_hi], axis=1)
    out1 = jnp.maximum(agg * dinv + b1_ref[...], 0.0)
    h2_ref[...] = jnp.dot(out1, w2_ref[...], preferred_element_type=_F32) * dinv


def _m3_body(a0, a1, h2, dega, degb, b2_ref, out_ref):
    dinv = _dinv_block(dega, degb)
    agg = a0[...] + a1[...] - h2[...]
    out_ref[...] = jnp.maximum(agg * dinv + b2_ref[...], 0.0)


def _row_spec(w):
    return pl.BlockSpec((_BLK, w), lambda i: (i, 0))


def _full_spec(shape):
    return pl.BlockSpec(shape, lambda i: tuple(0 for _ in shape))


def _m1(xp, W1, deg_a, deg_b):
    return pl.pallas_call(
        _m1_body,
        grid=_GRID,
        in_specs=[_row_spec(C_IN), _full_spec((C_IN, C_HID)),
                  _row_spec(16), _row_spec(16)],
        out_specs=[_row_spec(W128)] * 2,
        out_shape=[jax.ShapeDtypeStruct((N_PAD, W128), _F32)] * 2,
    )(xp, W1, deg_a, deg_b)


def _m2(alo0, alo1, ahi0, ahi1, hlo, hhi, deg_a, deg_b, b1, W2):
    return pl.pallas_call(
        _m2_body,
        grid=_GRID,
        in_specs=[_row_spec(W128)] * 6 +
                 [_row_spec(16), _row_spec(16),
                  _full_spec((1, C_HID)), _full_spec((C_HID, C_OUT))],
        out_specs=_row_spec(W128),
        out_shape=jax.ShapeDtypeStruct((N_PAD, W128), _F32),
    )(alo0, alo1, ahi0, ahi1, hlo, hhi, deg_a, deg_b,
      b1.reshape(1, C_HID), W2)


def _m3(a0, a1, h2, deg_a, deg_b, b2):
    return pl.pallas_call(
        _m3_body,
        grid=_GRID,
        in_specs=[_row_spec(W128)] * 3 +
                 [_row_spec(16), _row_spec(16), _full_spec((1, C_OUT))],
        out_specs=_row_spec(C_OUT),
        out_shape=jax.ShapeDtypeStruct((N_PAD, C_OUT), _F32),
    )(a0, a1, h2, deg_a, deg_b, b2.reshape(1, C_OUT))


def kernel(x, edge_index, W1, b1, W2, b2):
    src = edge_index[0]
    dst = edge_index[1]
    pad = E_PAD - E
    src1d = jnp.concatenate([src, jnp.zeros((pad,), jnp.int32)])
    dst_pad = jnp.concatenate([dst, jnp.full((pad,), TRASH_ROW, jnp.int32)])
    dst2d = dst_pad.reshape(E_PAD // 128, 128)
    xp = jnp.pad(x, ((0, N_PAD - N), (0, 0)))
    z16 = jnp.zeros((N_PAD, 16), _F32)

    deg_a, deg_b = _deg_kernel(dst2d, z16)
    hlo, hhi = _m1(xp, W1, deg_a, deg_b)
    alo0, alo1 = _agg_kernel(hlo, src1d, dst2d)
    ahi0, ahi1 = _agg_kernel(hhi, src1d, dst2d)
    h2 = _m2(alo0, alo1, ahi0, ahi1, hlo, hhi, deg_a, deg_b, b1, W2)
    a20, a21 = _agg_kernel(h2, src1d, dst2d)
    out = _m3(a20, a21, h2, deg_a, deg_b, b2)
    return out[:N]


# speedup vs baseline: 1.2321x; 1.2321x over previous
import functools

import jax
import jax.numpy as jnp
from jax import lax
from jax.experimental import pallas as pl
from jax.experimental.pallas import tpu as pltpu
from jax.experimental.pallas import tpu_sc as plsc

N = 10000
E = 320000
C_IN = 128
C_HID = 256
C_OUT = 128

NC = 2
NS = 16
NW = NC * NS
CHUNK = 128
W128 = 128

N_PAD = 10240
ROWS_PER_TILE = N_PAD // NS
TRASH_ROW = N

_EQ = NW * CHUNK * 8
E_PAD = ((E + _EQ - 1) // _EQ) * _EQ
WTILE_E = E_PAD // NW
WCHUNKS = WTILE_E // CHUNK
TOT_CHUNKS = E_PAD // CHUNK
A_CHUNKS = 128
B_CHUNKS = TOT_CHUNKS // NS - A_CHUNKS
SEG_CHUNKS = 32

_F32 = jnp.float32


def _mesh():
    return plsc.VectorSubcoreMesh(core_axis_name="c", subcore_axis_name="s")


@functools.partial(
    pl.kernel,
    out_type=(
        jax.ShapeDtypeStruct((N_PAD, 16), _F32),
        jax.ShapeDtypeStruct((N_PAD, 16), _F32),
    ),
    mesh=_mesh(),
    scratch_types=[
        pltpu.VMEM((CHUNK, 16), _F32),
        pltpu.VMEM((WCHUNKS, 128), jnp.int32),
        pltpu.VMEM_SHARED((N_PAD, 16), _F32),
    ],
)
def _deg_kernel(dst2d, z16, deg_a, deg_b, ones_v, didx, dacc):
    c = lax.axis_index("c")
    s = lax.axis_index("s")
    wid = s * NC + c
    rows = pl.ds(s * ROWS_PER_TILE, ROWS_PER_TILE)

    pltpu.sync_copy(z16.at[rows], dacc.at[rows])

    def _fill(i, _):
        ones_v[i, :] = jnp.full((16,), 1.0, _F32)
        return 0

    lax.fori_loop(0, CHUNK, _fill, 0)
    pltpu.sync_copy(dst2d.at[pl.ds(wid * WCHUNKS, WCHUNKS)], didx)
    plsc.subcore_barrier()

    def _body(j, _):
        pltpu.sync_copy(ones_v, dacc.at[didx.at[j]], add=True)
        return 0

    lax.fori_loop(0, WCHUNKS, _body, 0)
    plsc.subcore_barrier()

    @pl.when(c == 0)
    def _():
        pltpu.sync_copy(dacc.at[rows], deg_a.at[rows])

    @pl.when(c == 1)
    def _():
        pltpu.sync_copy(dacc.at[rows], deg_b.at[rows])


@functools.partial(
    pl.kernel,
    out_type=(
        jax.ShapeDtypeStruct((N_PAD, W128), _F32),
        jax.ShapeDtypeStruct((N_PAD, W128), _F32),
    ),
    mesh=_mesh(),
    scratch_types=[
        pltpu.VMEM((SEG_CHUNKS * CHUNK,), jnp.int32),
        pltpu.VMEM((SEG_CHUNKS, 128), jnp.int32),
        pltpu.VMEM((CHUNK, W128), _F32),
        pltpu.VMEM((CHUNK, W128), _F32),
        pltpu.VMEM_SHARED((N_PAD, W128), _F32),
        pltpu.SemaphoreType.DMA,
        pltpu.SemaphoreType.DMA,
    ],
)
def _agg_kernel(table, src1d, dst2d, part_a, part_b, sidx, didx, gbuf0, gbuf1,
                acc, semg, sems):
    gbuf = (gbuf0, gbuf1)
    c = lax.axis_index("c")
    s = lax.axis_index("s")
    rows = pl.ds(s * ROWS_PER_TILE, ROWS_PER_TILE)

    pltpu.sync_copy(table.at[rows], acc.at[rows])
    plsc.subcore_barrier()

    def _g_start(j, b):
        pltpu.async_copy(table.at[sidx.at[pl.ds(j * CHUNK, CHUNK)]],
                         gbuf[b], semg)

    def _g_wait(j, b):
        pltpu.make_async_copy(table.at[sidx.at[pl.ds(j * CHUNK, CHUNK)]],
                              gbuf[b], semg).wait()

    def _s_sync(j, b):
        pltpu.sync_copy(gbuf[b], acc.at[didx.at[j]], add=True)

    def _edge_loop(base_chunk, nchunks):
        for seg in range(nchunks // SEG_CHUNKS):
            cb = base_chunk + seg * SEG_CHUNKS
            pltpu.sync_copy(
                src1d.at[pl.ds(cb * CHUNK, SEG_CHUNKS * CHUNK)], sidx)
            pltpu.sync_copy(dst2d.at[pl.ds(cb, SEG_CHUNKS)], didx)

            def _body(p, _):
                j0 = 2 * p
                j1 = j0 + 1
                _g_start(j0, 0)
                _g_start(j1, 1)
                _g_wait(j0, 0)
                _s_sync(j0, 0)
                _g_wait(j1, 1)
                _s_sync(j1, 1)
                return 0

            lax.fori_loop(0, SEG_CHUNKS // 2, _body, 0)

    @pl.when(c == 0)
    def _():
        _edge_loop(s * A_CHUNKS, A_CHUNKS)

    @pl.when(c == 1)
    def _():
        _edge_loop(NS * A_CHUNKS + s * B_CHUNKS, B_CHUNKS)

    plsc.subcore_barrier()

    @pl.when(c == 0)
    def _():
        pltpu.sync_copy(acc.at[rows], part_a.at[rows])

    @pl.when(c == 1)
    def _():
        pltpu.sync_copy(acc.at[rows], part_b.at[rows])


_BLK = 256
_GRID = (N_PAD // _BLK,)


def _dinv_block(dega, degb):
    deg = dega[...] + degb[...] + 1.0
    return lax.rsqrt(deg)[:, 0:1]


def _m1_body(x_ref, w_ref, dega, degb, hlo_ref, hhi_ref):
    dinv = _dinv_block(dega, degb)
    h = jnp.dot(x_ref[...], w_ref[...], preferred_element_type=_F32) * dinv
    hlo_ref[...] = h[:, :W128]
    hhi_ref[...] = h[:, W128:]


def _m2_body(alo0, alo1, ahi0, ahi1, hlo, hhi, dega, degb, b1_ref, w2_ref,
             h2_ref):
    dinv = _dinv_block(dega, degb)
    agg_lo = alo0[...] + alo1[...] - hlo[...]
    agg_hi = ahi0[...] + ahi1[...] - hhi[...]
    agg = jnp.concatenate([agg_lo, agg_hi], axis=1)
    out1 = jnp.maximum(agg * dinv + b1_ref[...], 0.0)
    h2_ref[...] = jnp.dot(out1, w2_ref[...], preferred_element_type=_F32) * dinv


def _m3_body(a0, a1, h2, dega, degb, b2_ref, out_ref):
    dinv = _dinv_block(dega, degb)
    agg = a0[...] + a1[...] - h2[...]
    out_ref[...] = jnp.maximum(agg * dinv + b2_ref[...], 0.0)


def _row_spec(w):
    return pl.BlockSpec((_BLK, w), lambda i: (i, 0))


def _full_spec(shape):
    return pl.BlockSpec(shape, lambda i: tuple(0 for _ in shape))


def _m1(xp, W1, deg_a, deg_b):
    return pl.pallas_call(
        _m1_body,
        grid=_GRID,
        in_specs=[_row_spec(C_IN), _full_spec((C_IN, C_HID)),
                  _row_spec(16), _row_spec(16)],
        out_specs=[_row_spec(W128)] * 2,
        out_shape=[jax.ShapeDtypeStruct((N_PAD, W128), _F32)] * 2,
    )(xp, W1, deg_a, deg_b)


def _m2(alo0, alo1, ahi0, ahi1, hlo, hhi, deg_a, deg_b, b1, W2):
    return pl.pallas_call(
        _m2_body,
        grid=_GRID,
        in_specs=[_row_spec(W128)] * 6 +
                 [_row_spec(16), _row_spec(16),
                  _full_spec((1, C_HID)), _full_spec((C_HID, C_OUT))],
        out_specs=_row_spec(W128),
        out_shape=jax.ShapeDtypeStruct((N_PAD, W128), _F32),
    )(alo0, alo1, ahi0, ahi1, hlo, hhi, deg_a, deg_b,
      b1.reshape(1, C_HID), W2)


def _m3(a0, a1, h2, deg_a, deg_b, b2):
    return pl.pallas_call(
        _m3_body,
        grid=_GRID,
        in_specs=[_row_spec(W128)] * 3 +
                 [_row_spec(16), _row_spec(16), _full_spec((1, C_OUT))],
        out_specs=_row_spec(C_OUT),
        out_shape=jax.ShapeDtypeStruct((N_PAD, C_OUT), _F32),
    )(a0, a1, h2, deg_a, deg_b, b2.reshape(1, C_OUT))


def kernel(x, edge_index, W1, b1, W2, b2):
    src = edge_index[0]
    dst = edge_index[1]
    pad = E_PAD - E
    src1d = jnp.concatenate([src, jnp.zeros((pad,), jnp.int32)])
    dst_pad = jnp.concatenate([dst, jnp.full((pad,), TRASH_ROW, jnp.int32)])
    dst2d = dst_pad.reshape(E_PAD // 128, 128)
    xp = jnp.pad(x, ((0, N_PAD - N), (0, 0)))
    z16 = jnp.zeros((N_PAD, 16), _F32)

    deg_a, deg_b = _deg_kernel(dst2d, z16)
    hlo, hhi = _m1(xp, W1, deg_a, deg_b)
    alo0, alo1 = _agg_kernel(hlo, src1d, dst2d)
    ahi0, ahi1 = _agg_kernel(hhi, src1d, dst2d)
    h2 = _m2(alo0, alo1, ahi0, ahi1, hlo, hhi, deg_a, deg_b, b1, W2)
    a20, a21 = _agg_kernel(h2, src1d, dst2d)
    out = _m3(a20, a21, h2, deg_a, deg_b, b2)
    return out[:N]


# speedup vs baseline: 2.3937x; 1.9428x over previous
import functools

import jax
import jax.numpy as jnp
from jax import lax
from jax.experimental import pallas as pl
from jax.experimental.pallas import tpu as pltpu
from jax.experimental.pallas import tpu_sc as plsc

N = 10000
E = 320000
C_IN = 128
C_HID = 256
C_OUT = 128

NC = 2
NS = 16
NW = NC * NS
CHUNK = 128
W128 = 128

N_PAD = 10240
ROWS_PER_TILE = N_PAD // NS
TRASH_ROW = N

_EQ = NW * CHUNK * 8
E_PAD = ((E + _EQ - 1) // _EQ) * _EQ
WTILE_E = E_PAD // NW
WCHUNKS = WTILE_E // CHUNK
TOT_CHUNKS = E_PAD // CHUNK
A_CHUNKS = 136
B_CHUNKS = TOT_CHUNKS // NS - A_CHUNKS
SEG_CHUNKS = 32

_F32 = jnp.float32


def _mesh():
    return plsc.VectorSubcoreMesh(core_axis_name="c", subcore_axis_name="s")


@functools.partial(
    pl.kernel,
    out_type=(
        jax.ShapeDtypeStruct((N_PAD, 16), _F32),
        jax.ShapeDtypeStruct((N_PAD, 16), _F32),
    ),
    mesh=_mesh(),
    scratch_types=[
        pltpu.VMEM((CHUNK, 16), _F32),
        pltpu.VMEM((WCHUNKS, 128), jnp.int32),
        pltpu.VMEM_SHARED((N_PAD, 16), _F32),
    ],
)
def _deg_kernel(dst2d, z16, deg_a, deg_b, ones_v, didx, dacc):
    c = lax.axis_index("c")
    s = lax.axis_index("s")
    wid = s * NC + c
    rows = pl.ds(s * ROWS_PER_TILE, ROWS_PER_TILE)

    pltpu.sync_copy(z16.at[rows], dacc.at[rows])

    def _fill(i, _):
        ones_v[i, :] = jnp.full((16,), 1.0, _F32)
        return 0

    lax.fori_loop(0, CHUNK, _fill, 0)
    pltpu.sync_copy(dst2d.at[pl.ds(wid * WCHUNKS, WCHUNKS)], didx)
    plsc.subcore_barrier()

    def _body(j, _):
        pltpu.sync_copy(ones_v, dacc.at[didx.at[j]], add=True)
        return 0

    lax.fori_loop(0, WCHUNKS, _body, 0)
    plsc.subcore_barrier()

    @pl.when(c == 0)
    def _():
        pltpu.sync_copy(dacc.at[rows], deg_a.at[rows])

    @pl.when(c == 1)
    def _():
        pltpu.sync_copy(dacc.at[rows], deg_b.at[rows])


@functools.partial(
    pl.kernel,
    out_type=(
        jax.ShapeDtypeStruct((N_PAD, W128), _F32),
        jax.ShapeDtypeStruct((N_PAD, W128), _F32),
    ),
    mesh=_mesh(),
    scratch_types=[
        pltpu.VMEM((SEG_CHUNKS * CHUNK,), jnp.int32),
        pltpu.VMEM((SEG_CHUNKS, 128), jnp.int32),
        pltpu.VMEM((CHUNK, W128), _F32),
        pltpu.VMEM((CHUNK, W128), _F32),
        pltpu.VMEM_SHARED((N_PAD, W128), _F32),
        pltpu.SemaphoreType.DMA,
        pltpu.SemaphoreType.DMA,
    ],
)
def _agg_kernel(table, src1d, dst2d, part_a, part_b, sidx, didx, gbuf0, gbuf1,
                acc, semg, sems):
    gbuf = (gbuf0, gbuf1)
    c = lax.axis_index("c")
    s = lax.axis_index("s")
    rows = pl.ds(s * ROWS_PER_TILE, ROWS_PER_TILE)

    pltpu.sync_copy(table.at[rows], acc.at[rows])
    plsc.subcore_barrier()

    def _g_start(j, b):
        pltpu.async_copy(table.at[sidx.at[pl.ds(j * CHUNK, CHUNK)]],
                         gbuf[b], semg)

    def _g_wait(j, b):
        pltpu.make_async_copy(table.at[sidx.at[pl.ds(j * CHUNK, CHUNK)]],
                              gbuf[b], semg).wait()

    def _s_sync(j, b):
        pltpu.sync_copy(gbuf[b], acc.at[didx.at[j]], add=True)

    def _edge_loop(base_chunk, nchunks):
        for seg in range(nchunks // SEG_CHUNKS):
            cb = base_chunk + seg * SEG_CHUNKS
            pltpu.sync_copy(
                src1d.at[pl.ds(cb * CHUNK, SEG_CHUNKS * CHUNK)], sidx)
            pltpu.sync_copy(dst2d.at[pl.ds(cb, SEG_CHUNKS)], didx)

            def _body(p, _):
                j0 = 2 * p
                j1 = j0 + 1
                _g_start(j0, 0)
                _g_start(j1, 1)
                _g_wait(j0, 0)
                _s_sync(j0, 0)
                _g_wait(j1, 1)
                _s_sync(j1, 1)
                return 0

            lax.fori_loop(0, SEG_CHUNKS // 2, _body, 0)

    @pl.when(c == 0)
    def _():
        _edge_loop(s * A_CHUNKS, A_CHUNKS)

    @pl.when(c == 1)
    def _():
        _edge_loop(NS * A_CHUNKS + s * B_CHUNKS, B_CHUNKS)

    plsc.subcore_barrier()

    @pl.when(c == 0)
    def _():
        pltpu.sync_copy(acc.at[rows], part_a.at[rows])

    @pl.when(c == 1)
    def _():
        pltpu.sync_copy(acc.at[rows], part_b.at[rows])


_BLK = 256
_GRID = (N_PAD // _BLK,)


def _dinv_block(dega, degb):
    deg = dega[...] + degb[...] + 1.0
    return lax.rsqrt(deg)[:, 0:1]


def _m1_body(x_ref, w_ref, dega, degb, hlo_ref, hhi_ref):
    dinv = _dinv_block(dega, degb)
    h = jnp.dot(x_ref[...], w_ref[...], preferred_element_type=_F32) * dinv
    hlo_ref[...] = h[:, :W128]
    hhi_ref[...] = h[:, W128:]


def _m2_body(alo0, alo1, ahi0, ahi1, hlo, hhi, dega, degb, b1_ref, w2_ref,
             h2_ref):
    dinv = _dinv_block(dega, degb)
    agg_lo = alo0[...] + alo1[...] - hlo[...]
    agg_hi = ahi0[...] + ahi1[...] - hhi[...]
    agg = jnp.concatenate([agg_lo, agg_hi], axis=1)
    out1 = jnp.maximum(agg * dinv + b1_ref[...], 0.0)
    h2_ref[...] = jnp.dot(out1, w2_ref[...], preferred_element_type=_F32) * dinv


def _m3_body(a0, a1, h2, dega, degb, b2_ref, out_ref):
    dinv = _dinv_block(dega, degb)
    agg = a0[...] + a1[...] - h2[...]
    out_ref[...] = jnp.maximum(agg * dinv + b2_ref[...], 0.0)


def _row_spec(w):
    return pl.BlockSpec((_BLK, w), lambda i: (i, 0))


def _full_spec(shape):
    return pl.BlockSpec(shape, lambda i: tuple(0 for _ in shape))


def _m1(xp, W1, deg_a, deg_b):
    return pl.pallas_call(
        _m1_body,
        grid=_GRID,
        in_specs=[_row_spec(C_IN), _full_spec((C_IN, C_HID)),
                  _row_spec(16), _row_spec(16)],
        out_specs=[_row_spec(W128)] * 2,
        out_shape=[jax.ShapeDtypeStruct((N_PAD, W128), _F32)] * 2,
    )(xp, W1, deg_a, deg_b)


def _m2(alo0, alo1, ahi0, ahi1, hlo, hhi, deg_a, deg_b, b1, W2):
    return pl.pallas_call(
        _m2_body,
        grid=_GRID,
        in_specs=[_row_spec(W128)] * 6 +
                 [_row_spec(16), _row_spec(16),
                  _full_spec((1, C_HID)), _full_spec((C_HID, C_OUT))],
        out_specs=_row_spec(W128),
        out_shape=jax.ShapeDtypeStruct((N_PAD, W128), _F32),
    )(alo0, alo1, ahi0, ahi1, hlo, hhi, deg_a, deg_b,
      b1.reshape(1, C_HID), W2)


def _m3(a0, a1, h2, deg_a, deg_b, b2):
    return pl.pallas_call(
        _m3_body,
        grid=_GRID,
        in_specs=[_row_spec(W128)] * 3 +
                 [_row_spec(16), _row_spec(16), _full_spec((1, C_OUT))],
        out_specs=_row_spec(C_OUT),
        out_shape=jax.ShapeDtypeStruct((N_PAD, C_OUT), _F32),
    )(a0, a1, h2, deg_a, deg_b, b2.reshape(1, C_OUT))


def kernel(x, edge_index, W1, b1, W2, b2):
    src = edge_index[0]
    dst = edge_index[1]
    pad = E_PAD - E
    src1d = jnp.concatenate([src, jnp.zeros((pad,), jnp.int32)])
    dst_pad = jnp.concatenate([dst, jnp.full((pad,), TRASH_ROW, jnp.int32)])
    dst2d = dst_pad.reshape(E_PAD // 128, 128)
    xp = jnp.pad(x, ((0, N_PAD - N), (0, 0)))
    z16 = jnp.zeros((N_PAD, 16), _F32)

    deg_a, deg_b = _deg_kernel(dst2d, z16)
    hlo, hhi = _m1(xp, W1, deg_a, deg_b)
    alo0, alo1 = _agg_kernel(hlo, src1d, dst2d)
    ahi0, ahi1 = _agg_kernel(hhi, src1d, dst2d)
    h2 = _m2(alo0, alo1, ahi0, ahi1, hlo, hhi, deg_a, deg_b, b1, W2)
    a20, a21 = _agg_kernel(h2, src1d, dst2d)
    out = _m3(a20, a21, h2, deg_a, deg_b, b2)
    return out[:N]
